# baseline (speedup 1.0000x reference)
import functools

import jax
import jax.numpy as jnp
from jax import lax
from jax.experimental import pallas as pl
from jax.experimental.pallas import tpu as pltpu
from jax.experimental.pallas import tpu_sc as plsc

_C, _T, _H, _W = 3, 64, 512, 512
_ALPHA = 4
_TS = _T // _ALPHA
_NW = 32
_FRAMES_PER_WORKER = _C * _T // _NW


def _copy_body(x_ref, o_ref):
    o_ref[...] = x_ref[...]


_tc_slow_gather = pl.pallas_call(
    _copy_body,
    grid=(_C, _TS),
    in_specs=[
        pl.BlockSpec((1, 1, _H, _W), lambda c, p: (c, (21 * p) // 5, 0, 0))
    ],
    out_specs=pl.BlockSpec((1, 1, _H, _W), lambda c, p: (c, p, 0, 0)),
    out_shape=jax.ShapeDtypeStruct((_C, _TS, _H, _W), jnp.float32),
)


_NBUF = 3
_CROWS = 64
_CHUNKS_PER_FRAME = _H // _CROWS
_NCH = _FRAMES_PER_WORKER * _CHUNKS_PER_FRAME
_NSUPER = _NCH // _NBUF


@functools.partial(
    pl.kernel,
    mesh=plsc.VectorSubcoreMesh(core_axis_name="c", subcore_axis_name="s"),
    out_type=jax.ShapeDtypeStruct((_C, _T, _H, _W), jnp.float32),
    scratch_types=[
        [pltpu.VMEM((_CROWS, _W), jnp.float32)] * _NBUF,
        [pltpu.SemaphoreType.DMA] * _NBUF,
        [pltpu.SemaphoreType.DMA] * _NBUF,
    ],
)
def _sc_fast_copy(frames_hbm, out_hbm, bufs, rsems, wsems):
    wid = lax.axis_index("s") * 2 + lax.axis_index("c")
    base = wid * _NCH

    def _src(g):
        f = g // _CHUNKS_PER_FRAME
        r = (g % _CHUNKS_PER_FRAME) * _CROWS
        return frames_hbm.at[f // _T, f % _T, pl.ds(r, _CROWS), :]

    def _dst(g):
        f = g // _CHUNKS_PER_FRAME
        r = (g % _CHUNKS_PER_FRAME) * _CROWS
        return out_hbm.at[f // _T, f % _T, pl.ds(r, _CROWS), :]

    for j in range(_NBUF):
        pltpu.async_copy(_src(base + j), bufs[j], rsems[j])

    def body(it, carry):
        g0 = base + it * _NBUF
        for j in range(_NBUF):
            pltpu.make_async_copy(_src(g0 + j), bufs[j], rsems[j]).wait()
            pltpu.async_copy(bufs[j], _dst(g0 + j), wsems[j])

        @pl.when(it < _NSUPER - 1)
        def _():
            for j in range(_NBUF):
                pltpu.make_async_copy(bufs[j], _dst(g0 + j), wsems[j]).wait()
                pltpu.async_copy(_src(g0 + _NBUF + j), bufs[j], rsems[j])

        return carry

    lax.fori_loop(0, _NSUPER, body, 0)

    g_last = base + (_NSUPER - 1) * _NBUF
    for j in range(_NBUF):
        pltpu.make_async_copy(bufs[j], _dst(g_last + j), wsems[j]).wait()


def kernel(frames):
    fast = _sc_fast_copy(frames)
    slow = _tc_slow_gather(frames)
    return (slow, fast)


# speedup vs baseline: 1.0171x; 1.0171x over previous
import functools

import jax
import jax.numpy as jnp
from jax import lax
from jax.experimental import pallas as pl
from jax.experimental.pallas import tpu as pltpu
from jax.experimental.pallas import tpu_sc as plsc

_C, _T, _H, _W = 3, 64, 512, 512
_ALPHA = 4
_TS = _T // _ALPHA
_NW = 32
_FRAMES_PER_WORKER = _C * _T // _NW


def _copy_body(x_ref, o_ref):
    o_ref[...] = x_ref[...]


_tc_slow_gather = pl.pallas_call(
    _copy_body,
    grid=(_C, _TS),
    in_specs=[
        pl.BlockSpec((1, 1, _H, _W), lambda c, p: (c, (21 * p) // 5, 0, 0))
    ],
    out_specs=pl.BlockSpec((1, 1, _H, _W), lambda c, p: (c, p, 0, 0)),
    out_shape=jax.ShapeDtypeStruct((_C, _TS, _H, _W), jnp.float32),
)


_NBUF = 6
_CROWS = 32
_CHUNKS_PER_FRAME = _H // _CROWS
_NCH = _FRAMES_PER_WORKER * _CHUNKS_PER_FRAME
_NSUPER = _NCH // _NBUF


@functools.partial(
    pl.kernel,
    mesh=plsc.VectorSubcoreMesh(core_axis_name="c", subcore_axis_name="s"),
    out_type=jax.ShapeDtypeStruct((_C, _T, _H, _W), jnp.float32),
    scratch_types=[
        [pltpu.VMEM((_CROWS, _W), jnp.float32)] * _NBUF,
        [pltpu.SemaphoreType.DMA] * _NBUF,
        [pltpu.SemaphoreType.DMA] * _NBUF,
    ],
)
def _sc_fast_copy(frames_hbm, out_hbm, bufs, rsems, wsems):
    wid = lax.axis_index("s") * 2 + lax.axis_index("c")
    base = wid * _NCH

    def _src(g):
        f = g // _CHUNKS_PER_FRAME
        r = (g % _CHUNKS_PER_FRAME) * _CROWS
        return frames_hbm.at[f // _T, f % _T, pl.ds(r, _CROWS), :]

    def _dst(g):
        f = g // _CHUNKS_PER_FRAME
        r = (g % _CHUNKS_PER_FRAME) * _CROWS
        return out_hbm.at[f // _T, f % _T, pl.ds(r, _CROWS), :]

    for j in range(_NBUF):
        pltpu.async_copy(_src(base + j), bufs[j], rsems[j])

    def body(it, carry):
        g0 = base + it * _NBUF
        for j in range(_NBUF):
            pltpu.make_async_copy(_src(g0 + j), bufs[j], rsems[j]).wait()
            pltpu.async_copy(bufs[j], _dst(g0 + j), wsems[j])

        @pl.when(it < _NSUPER - 1)
        def _():
            for j in range(_NBUF):
                pltpu.make_async_copy(bufs[j], _dst(g0 + j), wsems[j]).wait()
                pltpu.async_copy(_src(g0 + _NBUF + j), bufs[j], rsems[j])

        return carry

    lax.fori_loop(0, _NSUPER, body, 0)

    g_last = base + (_NSUPER - 1) * _NBUF
    for j in range(_NBUF):
        pltpu.make_async_copy(bufs[j], _dst(g_last + j), wsems[j]).wait()


def kernel(frames):
    fast = _sc_fast_copy(frames)
    slow = _tc_slow_gather(frames)
    return (slow, fast)


# speedup vs baseline: 1.1125x; 1.0938x over previous
import functools

import jax
import jax.numpy as jnp
from jax import lax
from jax.experimental import pallas as pl
from jax.experimental.pallas import tpu as pltpu
from jax.experimental.pallas import tpu_sc as plsc

_C, _T, _H, _W = 3, 64, 512, 512
_ALPHA = 4
_TS = _T // _ALPHA
_NW = 32


def _copy_body(x_ref, o_ref):
    o_ref[...] = x_ref[...]


_tc_fast_copy = pl.pallas_call(
    _copy_body,
    grid=(_C, _T // 8),
    in_specs=[pl.BlockSpec((1, 8, _H, _W), lambda c, i: (c, i, 0, 0))],
    out_specs=pl.BlockSpec((1, 8, _H, _W), lambda c, i: (c, i, 0, 0)),
    out_shape=jax.ShapeDtypeStruct((_C, _T, _H, _W), jnp.float32),
)


def _make_sc_ring_copy(out_shape, nbuf, crows, chunks_total, src_at, dst_at):
    nch = chunks_total // _NW
    nsuper = nch // nbuf

    @functools.partial(
        pl.kernel,
        mesh=plsc.VectorSubcoreMesh(core_axis_name="c", subcore_axis_name="s"),
        out_type=jax.ShapeDtypeStruct(out_shape, jnp.float32),
        scratch_types=[
            [pltpu.VMEM((crows, _W), jnp.float32)] * nbuf,
            [pltpu.SemaphoreType.DMA] * nbuf,
            [pltpu.SemaphoreType.DMA] * nbuf,
        ],
    )
    def sc_copy(in_hbm, out_hbm, bufs, rsems, wsems):
        wid = lax.axis_index("s") * 2 + lax.axis_index("c")
        base = wid * nch

        for j in range(nbuf):
            pltpu.async_copy(src_at(in_hbm, base + j), bufs[j], rsems[j])

        def body(it, carry):
            g0 = base + it * nbuf
            for j in range(nbuf):
                pltpu.make_async_copy(
                    src_at(in_hbm, g0 + j), bufs[j], rsems[j]
                ).wait()
                pltpu.async_copy(bufs[j], dst_at(out_hbm, g0 + j), wsems[j])

            @pl.when(it < nsuper - 1)
            def _():
                for j in range(nbuf):
                    pltpu.make_async_copy(
                        bufs[j], dst_at(out_hbm, g0 + j), wsems[j]
                    ).wait()
                    pltpu.async_copy(
                        src_at(in_hbm, g0 + nbuf + j), bufs[j], rsems[j]
                    )

            return carry

        lax.fori_loop(0, nsuper, body, 0)

        g_last = base + (nsuper - 1) * nbuf
        for j in range(nbuf):
            pltpu.make_async_copy(
                bufs[j], dst_at(out_hbm, g_last + j), wsems[j]
            ).wait()

    return sc_copy


_G_CROWS = 32
_G_CPF = _H // _G_CROWS
_G_TOTAL = _C * _TS * _G_CPF


def _gather_src(ref, g):
    j = g // _G_CPF
    r = (g % _G_CPF) * _G_CROWS
    c = j // _TS
    t = (21 * (j % _TS)) // 5
    return ref.at[c, t, pl.ds(r, _G_CROWS), :]


def _gather_dst(ref, g):
    j = g // _G_CPF
    r = (g % _G_CPF) * _G_CROWS
    return ref.at[j // _TS, j % _TS, pl.ds(r, _G_CROWS), :]


_sc_slow_gather = _make_sc_ring_copy(
    (_C, _TS, _H, _W), 4, _G_CROWS, _G_TOTAL, _gather_src, _gather_dst
)


def kernel(frames):
    slow = _sc_slow_gather(frames)
    fast = _tc_fast_copy(frames)
    return (slow, fast)
